# baseline (speedup 1.0000x reference)
import jax
import jax.numpy as jnp
from jax import lax
from jax.experimental import pallas as pl
from jax.experimental.pallas import tpu as pltpu
from jax.experimental.pallas import tpu_sc as plsc

N = 10000
E = 320000
D = 128

NUM_CORES = 2
NUM_SUBCORES = 16
NUM_WORKERS = NUM_CORES * NUM_SUBCORES
EDGES_PER_WORKER = E // NUM_WORKERS
CHUNK = 80
NUM_CHUNKS = EDGES_PER_WORKER // CHUNK
NPAD = 10240
ROWS_PER_TILE = NPAD // NUM_SUBCORES


def _spmm_body(x_hbm, edge_hbm, w_hbm, out_hbm,
               acc, src_v, dst0, dst1, dst2, w0, w1, w2, rows0, rows1, rows2,
               sem0, sem1, sem2, ssem0, ssem1, ssem2):
    cid = lax.axis_index("core")
    sid = lax.axis_index("subcore")
    wid = cid * NUM_SUBCORES + sid
    rbufs = (rows0, rows1, rows2)
    dbufs = (dst0, dst1, dst2)
    wbufs = (w0, w1, w2)
    sems = (sem0, sem1, sem2)
    ssems = (ssem0, ssem1, ssem2)
    base0 = wid * EDGES_PER_WORKER

    pltpu.make_async_copy(edge_hbm.at[pl.ds(E + base0, EDGES_PER_WORKER)],
                          src_v, sem0).start()

    @pl.loop(0, CHUNK)
    def _(i):
        for k in range(D // 16):
            rows2[i, pl.ds(k * 16, 16)] = jnp.zeros((16,), jnp.float32)

    pltpu.make_async_copy(edge_hbm.at[pl.ds(E + base0, EDGES_PER_WORKER)],
                          src_v, sem0).wait()

    def chunk_copies(j, b):
        base = base0 + j * CHUNK
        return (
            pltpu.make_async_copy(x_hbm.at[src_v.at[pl.ds(j * CHUNK, CHUNK)]],
                                  rbufs[b], sems[b]),
            pltpu.make_async_copy(edge_hbm.at[pl.ds(base, CHUNK)],
                                  dbufs[b], sems[b]),
            pltpu.make_async_copy(w_hbm.at[pl.ds(base, CHUNK)], wbufs[b], sems[b]),
        )

    def start(j, b):
        for cp in chunk_copies(j, b):
            cp.start()

    def scatter_cp(b):
        return pltpu.make_async_copy(rbufs[b], acc.at[dbufs[b]], ssems[b])

    def step(j, b, prev_wait, fire):
        if fire:
            if prev_wait:
                scatter_cp((b + 2) % 3).wait()
            start(j + 2, (b + 2) % 3)
        for cp in chunk_copies(j, b):
            cp.wait()
        rows, wv = rbufs[b], wbufs[b]

        @plsc.parallel_loop(0, CHUNK // 16)
        def _(g):
            wg = wv[pl.ds(g * 16, 16)]
            for e in range(16):
                we = wg[e]
                r = g * 16 + e
                for k in range(D // 16):
                    sl = pl.ds(k * 16, 16)
                    rows[r, sl] = rows[r, sl] * we

        scatter_cp(b).start(add=True)

    start(0, 0)
    start(1, 1)
    row0 = sid * ROWS_PER_TILE
    for z in range(ROWS_PER_TILE // CHUNK):
        pltpu.sync_copy(rows2, acc.at[pl.ds(row0 + z * CHUNK, CHUNK)])
    plsc.subcore_barrier()

    step(0, 0, False, True)
    step(1, 1, True, True)
    step(2, 2, True, True)

    @pl.loop(3, NUM_CHUNKS - 2, step=3)
    def _(j):
        step(j, 0, True, True)
        step(j + 1, 1, True, True)
        step(j + 2, 2, True, True)

    step(NUM_CHUNKS - 2, 0, False, False)
    step(NUM_CHUNKS - 1, 1, False, False)
    scatter_cp(2).wait()
    scatter_cp(0).wait()
    scatter_cp(1).wait()

    plsc.subcore_barrier()

    pltpu.sync_copy(acc.at[pl.ds(row0, ROWS_PER_TILE)],
                    out_hbm.at[cid].at[pl.ds(row0, ROWS_PER_TILE)])


def _sc_spmm(x, edge_index, w):
    mesh = plsc.VectorSubcoreMesh(core_axis_name="core", subcore_axis_name="subcore")
    kern = pl.kernel(
        _spmm_body,
        out_type=jax.ShapeDtypeStruct((NUM_CORES, NPAD, D), jnp.float32),
        mesh=mesh,
        scratch_types=[
            pltpu.VMEM_SHARED((NPAD, D), jnp.float32),
            pltpu.VMEM((EDGES_PER_WORKER,), jnp.int32),
            pltpu.VMEM((CHUNK,), jnp.int32),
            pltpu.VMEM((CHUNK,), jnp.int32),
            pltpu.VMEM((CHUNK,), jnp.int32),
            pltpu.VMEM((CHUNK,), jnp.float32),
            pltpu.VMEM((CHUNK,), jnp.float32),
            pltpu.VMEM((CHUNK,), jnp.float32),
            pltpu.VMEM((CHUNK, D), jnp.float32),
            pltpu.VMEM((CHUNK, D), jnp.float32),
            pltpu.VMEM((CHUNK, D), jnp.float32),
            pltpu.SemaphoreType.DMA,
            pltpu.SemaphoreType.DMA,
            pltpu.SemaphoreType.DMA,
            pltpu.SemaphoreType.DMA,
            pltpu.SemaphoreType.DMA,
            pltpu.SemaphoreType.DMA,
        ],
    )
    return kern(x, edge_index, w)


def _v0_body(x_ref, w0_ref, off0_ref, sc0_ref, v0_ref):
    vw0 = jnp.dot(x_ref[...], w0_ref[...], preferred_element_type=jnp.float32)
    v0_ref[...] = _bn_relu(vw0, sc0_ref[...], off0_ref[...])


def _tc_v0(x, W0, offset0, scale0):
    blk = 1000
    row_spec = pl.BlockSpec((blk, D), lambda i: (i, 0))
    full = pl.BlockSpec((D, D), lambda i: (0, 0))
    vec = pl.BlockSpec((1, D), lambda i: (0, 0))
    return pl.pallas_call(
        _v0_body,
        grid=(N // blk,),
        in_specs=[row_spec, full, vec, vec],
        out_specs=row_spec,
        out_shape=jax.ShapeDtypeStruct((N, D), jnp.float32),
    )(x, W0, offset0, scale0)


def _bn_relu(vw, scale, offset):
    mean = jnp.mean(vw, axis=1, keepdims=True)
    cent = vw - mean
    var = jnp.mean(cent * cent, axis=1, keepdims=True)
    return jnp.maximum(scale * cent * lax.rsqrt(var + 1e-9) + offset, 0.0)


def _v1_body(v0_ref, p0_ref, p1_ref, w1_ref, off1_ref, sc1_ref, out_ref):
    h1 = p0_ref[0] + p1_ref[0]
    vw1 = jnp.dot(h1, w1_ref[...], preferred_element_type=jnp.float32)
    out_ref[...] = v0_ref[...] + _bn_relu(vw1, sc1_ref[...], off1_ref[...])


def _tc_v1(v0, partials, W1, offset1, scale1):
    blk = 2000
    row_spec = pl.BlockSpec((blk, D), lambda i: (i, 0))
    full = pl.BlockSpec((D, D), lambda i: (0, 0))
    vec = pl.BlockSpec((1, D), lambda i: (0, 0))
    return pl.pallas_call(
        _v1_body,
        grid=(N // blk,),
        in_specs=[row_spec,
                  pl.BlockSpec((1, blk, D), lambda i: (0, i, 0)),
                  pl.BlockSpec((1, blk, D), lambda i: (1, i, 0)),
                  full, vec, vec],
        out_specs=row_spec,
        out_shape=jax.ShapeDtypeStruct((N, D), jnp.float32),
    )(v0, partials, partials, W1, offset1, scale1)


def kernel(x, edge_index, edge_values, W0, W1, offset0, scale0, offset1, scale1):
    edges_flat = edge_index.astype(jnp.int32).reshape(-1)
    partials = _sc_spmm(x, edges_flat, edge_values)
    v0 = _tc_v0(x, W0, offset0, scale0)
    return _tc_v1(v0, partials, W1, offset1, scale1)


# speedup vs baseline: 1.2137x; 1.2137x over previous
import jax
import jax.numpy as jnp
from jax import lax
from jax.experimental import pallas as pl
from jax.experimental.pallas import tpu as pltpu
from jax.experimental.pallas import tpu_sc as plsc

N = 10000
E = 320000
D = 128

NUM_CORES = 2
NUM_SUBCORES = 16
NUM_WORKERS = NUM_CORES * NUM_SUBCORES
EDGES_PER_WORKER = E // NUM_WORKERS
CHUNK = 80
NUM_CHUNKS = EDGES_PER_WORKER // CHUNK
NPAD = 10240
ROWS_PER_TILE = NPAD // NUM_SUBCORES


def _spmm_body(x_hbm, edge_hbm, w_hbm, out_hbm,
               acc, src_v, dst0, dst1, dst2, w0, w1, w2, rows0, rows1, rows2,
               sem0, sem1, sem2, ssem0, ssem1, ssem2):
    cid = lax.axis_index("core")
    sid = lax.axis_index("subcore")
    wid = cid * NUM_SUBCORES + sid
    rbufs = (rows0, rows1, rows2)
    dbufs = (dst0, dst1, dst2)
    wbufs = (w0, w1, w2)
    sems = (sem0, sem1, sem2)
    ssems = (ssem0, ssem1, ssem2)
    base0 = wid * EDGES_PER_WORKER

    pltpu.make_async_copy(edge_hbm.at[pl.ds(E + base0, EDGES_PER_WORKER)],
                          src_v, sem0).start()

    @pl.loop(0, CHUNK)
    def _(i):
        for k in range(D // 16):
            rows2[i, pl.ds(k * 16, 16)] = jnp.zeros((16,), jnp.float32)

    pltpu.make_async_copy(edge_hbm.at[pl.ds(E + base0, EDGES_PER_WORKER)],
                          src_v, sem0).wait()

    def chunk_copies(j, b):
        base = base0 + j * CHUNK
        return (
            pltpu.make_async_copy(x_hbm.at[src_v.at[pl.ds(j * CHUNK, CHUNK)]],
                                  rbufs[b], sems[b]),
            pltpu.make_async_copy(edge_hbm.at[pl.ds(base, CHUNK)],
                                  dbufs[b], sems[b]),
            pltpu.make_async_copy(w_hbm.at[pl.ds(base, CHUNK)], wbufs[b], sems[b]),
        )

    def start(j, b):
        for cp in chunk_copies(j, b):
            cp.start()

    def scatter_cp(b):
        return pltpu.make_async_copy(rbufs[b], acc.at[dbufs[b]], ssems[b])

    def step(j, b, prev_wait, fire):
        if fire:
            if prev_wait:
                scatter_cp((b + 2) % 3).wait()
            start(j + 2, (b + 2) % 3)
        for cp in chunk_copies(j, b):
            cp.wait()
        rows, wv = rbufs[b], wbufs[b]

        @pl.loop(0, CHUNK // 16)
        def _(g):
            wg = wv[pl.ds(g * 16, 16)]
            for e in range(16):
                we = wg[e]
                r = g * 16 + e
                for k in range(D // 16):
                    sl = pl.ds(k * 16, 16)
                    rows[r, sl] = rows[r, sl] * we

        scatter_cp(b).start(add=True)

    start(0, 0)
    start(1, 1)
    row0 = sid * ROWS_PER_TILE
    for z in range(ROWS_PER_TILE // CHUNK):
        pltpu.sync_copy(rows2, acc.at[pl.ds(row0 + z * CHUNK, CHUNK)])
    plsc.subcore_barrier()

    step(0, 0, False, True)
    step(1, 1, True, True)
    step(2, 2, True, True)

    @pl.loop(3, NUM_CHUNKS - 2, step=3)
    def _(j):
        step(j, 0, True, True)
        step(j + 1, 1, True, True)
        step(j + 2, 2, True, True)

    step(NUM_CHUNKS - 2, 0, False, False)
    step(NUM_CHUNKS - 1, 1, False, False)
    scatter_cp(2).wait()
    scatter_cp(0).wait()
    scatter_cp(1).wait()

    plsc.subcore_barrier()

    pltpu.sync_copy(acc.at[pl.ds(row0, ROWS_PER_TILE)],
                    out_hbm.at[cid].at[pl.ds(row0, ROWS_PER_TILE)])


def _sc_spmm(x, edge_index, w):
    mesh = plsc.VectorSubcoreMesh(core_axis_name="core", subcore_axis_name="subcore")
    kern = pl.kernel(
        _spmm_body,
        out_type=jax.ShapeDtypeStruct((NUM_CORES, NPAD, D), jnp.float32),
        mesh=mesh,
        scratch_types=[
            pltpu.VMEM_SHARED((NPAD, D), jnp.float32),
            pltpu.VMEM((EDGES_PER_WORKER,), jnp.int32),
            pltpu.VMEM((CHUNK,), jnp.int32),
            pltpu.VMEM((CHUNK,), jnp.int32),
            pltpu.VMEM((CHUNK,), jnp.int32),
            pltpu.VMEM((CHUNK,), jnp.float32),
            pltpu.VMEM((CHUNK,), jnp.float32),
            pltpu.VMEM((CHUNK,), jnp.float32),
            pltpu.VMEM((CHUNK, D), jnp.float32),
            pltpu.VMEM((CHUNK, D), jnp.float32),
            pltpu.VMEM((CHUNK, D), jnp.float32),
            pltpu.SemaphoreType.DMA,
            pltpu.SemaphoreType.DMA,
            pltpu.SemaphoreType.DMA,
            pltpu.SemaphoreType.DMA,
            pltpu.SemaphoreType.DMA,
            pltpu.SemaphoreType.DMA,
        ],
    )
    return kern(x, edge_index, w)


def _v0_body(x_ref, w0_ref, off0_ref, sc0_ref, v0_ref):
    vw0 = jnp.dot(x_ref[...], w0_ref[...], preferred_element_type=jnp.float32)
    v0_ref[...] = _bn_relu(vw0, sc0_ref[...], off0_ref[...])


def _tc_v0(x, W0, offset0, scale0):
    blk = 1000
    row_spec = pl.BlockSpec((blk, D), lambda i: (i, 0))
    full = pl.BlockSpec((D, D), lambda i: (0, 0))
    vec = pl.BlockSpec((1, D), lambda i: (0, 0))
    return pl.pallas_call(
        _v0_body,
        grid=(N // blk,),
        in_specs=[row_spec, full, vec, vec],
        out_specs=row_spec,
        out_shape=jax.ShapeDtypeStruct((N, D), jnp.float32),
    )(x, W0, offset0, scale0)


def _bn_relu(vw, scale, offset):
    mean = jnp.mean(vw, axis=1, keepdims=True)
    cent = vw - mean
    var = jnp.mean(cent * cent, axis=1, keepdims=True)
    return jnp.maximum(scale * cent * lax.rsqrt(var + 1e-9) + offset, 0.0)


def _v1_body(v0_ref, p0_ref, p1_ref, w1_ref, off1_ref, sc1_ref, out_ref):
    h1 = p0_ref[0] + p1_ref[0]
    vw1 = jnp.dot(h1, w1_ref[...], preferred_element_type=jnp.float32)
    out_ref[...] = v0_ref[...] + _bn_relu(vw1, sc1_ref[...], off1_ref[...])


def _tc_v1(v0, partials, W1, offset1, scale1):
    blk = 2000
    row_spec = pl.BlockSpec((blk, D), lambda i: (i, 0))
    full = pl.BlockSpec((D, D), lambda i: (0, 0))
    vec = pl.BlockSpec((1, D), lambda i: (0, 0))
    return pl.pallas_call(
        _v1_body,
        grid=(N // blk,),
        in_specs=[row_spec,
                  pl.BlockSpec((1, blk, D), lambda i: (0, i, 0)),
                  pl.BlockSpec((1, blk, D), lambda i: (1, i, 0)),
                  full, vec, vec],
        out_specs=row_spec,
        out_shape=jax.ShapeDtypeStruct((N, D), jnp.float32),
    )(v0, partials, partials, W1, offset1, scale1)


def kernel(x, edge_index, edge_values, W0, W1, offset0, scale0, offset1, scale1):
    edges_flat = edge_index.astype(jnp.int32).reshape(-1)
    partials = _sc_spmm(x, edges_flat, edge_values)
    v0 = _tc_v0(x, W0, offset0, scale0)
    return _tc_v1(v0, partials, W1, offset1, scale1)
